# baseline (speedup 1.0000x reference)
import functools

import jax
import jax.numpy as jnp
from jax import lax
from jax.experimental import pallas as pl
from jax.experimental.pallas import tpu as pltpu
from jax.experimental.pallas import tpu_sc as plsc

_NUM_TOKENS = 32
_DIM = 768
_BATCH = 16384
_LANES = 16

_INFO = plsc.get_sparse_core_info()
_NC = _INFO.num_cores
_NS = _INFO.num_subcores
_NW = _NC * _NS
_BPW = _BATCH // _NW
_NGROUPS = _BPW // _LANES
_LAG = 8


_NSEM = 4
_SGRP = _NGROUPS // _NSEM
_SLAG = 3


def _body(tokens_hbm, idx_hbm, out_hbm, table_v, idx_v, st, si,
          s0, s1, s2, s3):
    wid = lax.axis_index("s") * _NC + lax.axis_index("c")
    base = wid * _BPW

    tcopy = pltpu.async_copy(tokens_hbm, table_v, st)
    icopy = pltpu.async_copy(idx_hbm.at[pl.ds(base, _BPW)], idx_v, si)
    icopy.wait()
    tcopy.wait()

    sems = (s0, s1, s2, s3)

    def super_group(sg, _):
        for q in range(_NSEM):
            g = sg * _NSEM + q
            idxv = idx_v[pl.ds(g * _LANES, _LANES)]

            @pl.when(sg >= _SLAG)
            def _():
                pltpu.make_async_copy(
                    table_v.at[pl.ds(0, _LANES)],
                    out_hbm.at[pl.ds(0, _LANES)], sems[q]).wait()

            for r in range(_LANES):
                tok = idxv[r]
                pltpu.async_copy(
                    table_v.at[tok], out_hbm.at[base + g * _LANES + r],
                    sems[q])
        return _

    lax.fori_loop(0, _SGRP, super_group, None)

    for q in range(_NSEM):
        for _ in range(_SLAG):
            pltpu.make_async_copy(
                table_v.at[pl.ds(0, _LANES)],
                out_hbm.at[pl.ds(0, _LANES)], sems[q]).wait()


_lookup = functools.partial(
    pl.kernel,
    out_type=jax.ShapeDtypeStruct((_BATCH, _DIM), jnp.float32),
    mesh=plsc.VectorSubcoreMesh(core_axis_name="c", subcore_axis_name="s"),
    scratch_types=[
        pltpu.VMEM((_NUM_TOKENS, _DIM), jnp.float32),
        pltpu.VMEM((_BPW,), jnp.int32),
        pltpu.SemaphoreType.DMA,
        pltpu.SemaphoreType.DMA,
        pltpu.SemaphoreType.DMA,
        pltpu.SemaphoreType.DMA,
        pltpu.SemaphoreType.DMA,
        pltpu.SemaphoreType.DMA,
    ],
)(_body)


@jax.jit
def kernel(tokens, indices):
    return _lookup(tokens, indices)


# speedup vs baseline: 1.0338x; 1.0338x over previous
import functools

import jax
import jax.numpy as jnp
from jax import lax
from jax.experimental import pallas as pl
from jax.experimental.pallas import tpu as pltpu
from jax.experimental.pallas import tpu_sc as plsc

_NUM_TOKENS = 32
_DIM = 768
_BATCH = 16384
_LANES = 16

_INFO = plsc.get_sparse_core_info()
_NC = _INFO.num_cores
_NS = _INFO.num_subcores
_NW = _NC * _NS
_BPW = _BATCH // _NW
_NGROUPS = _BPW // _LANES
_LAG = 8


def _body(tokens_hbm, idx_hbm, out_hbm, table_v, idx_v, st, si, sem):
    wid = lax.axis_index("s") * _NC + lax.axis_index("c")
    base = wid * _BPW

    tcopy = pltpu.async_copy(tokens_hbm, table_v, st)
    icopy = pltpu.async_copy(idx_hbm.at[pl.ds(base, _BPW)], idx_v, si)
    icopy.wait()
    tcopy.wait()

    def group(g, _):
        idxv = idx_v[pl.ds(g * _LANES, _LANES)]
        for r in range(_LANES):
            tok = idxv[r]
            pltpu.async_copy(
                table_v.at[tok], out_hbm.at[base + g * _LANES + r], sem)

        @pl.when(g >= _LAG)
        def _():
            pltpu.make_async_copy(
                table_v.at[pl.ds(0, _LANES)],
                out_hbm.at[pl.ds(0, _LANES)], sem).wait()

        return _

    lax.fori_loop(0, _NGROUPS, group, None)

    def drain(i, _):
        pltpu.make_async_copy(
            table_v.at[pl.ds(0, _LANES)],
            out_hbm.at[pl.ds(0, _LANES)], sem).wait()
        return _

    lax.fori_loop(0, _LAG, drain, None)


_lookup = functools.partial(
    pl.kernel,
    out_type=jax.ShapeDtypeStruct((_BATCH, _DIM), jnp.float32),
    mesh=plsc.VectorSubcoreMesh(core_axis_name="c", subcore_axis_name="s"),
    scratch_types=[
        pltpu.VMEM((_NUM_TOKENS, _DIM), jnp.float32),
        pltpu.VMEM((_BPW,), jnp.int32),
        pltpu.SemaphoreType.DMA,
        pltpu.SemaphoreType.DMA,
        pltpu.SemaphoreType.DMA,
    ],
)(_body)


@jax.jit
def kernel(tokens, indices):
    return _lookup(tokens, indices)
